# baseline (speedup 1.0000x reference)
import functools

import jax
import jax.numpy as jnp
from jax import lax
from jax.experimental import pallas as pl
from jax.experimental.pallas import tpu as pltpu
from jax.experimental.pallas import tpu_sc as plsc

B_ROWS = 16384
B_COLS = 26
DIM = 64
PDIM = 128
STRIPE = 128
N_STRIPES = B_ROWS // STRIPE

_info = plsc.get_sparse_core_info()
NC = _info.num_cores
NS = _info.num_subcores
NW = NC * NS
STRIPES_PER_W = N_STRIPES // NW
NBUF = 2

_mesh = plsc.VectorSubcoreMesh(core_axis_name="c", subcore_axis_name="s")


@functools.partial(
    pl.kernel,
    out_type=jax.ShapeDtypeStruct((B_COLS, 8, STRIPE, 8, PDIM), jnp.float32),
    mesh=_mesh,
    scratch_types=[
        pltpu.VMEM((STRIPE, B_COLS), jnp.int32),
        pltpu.VMEM((STRIPE,), jnp.int32),
        pltpu.VMEM((STRIPE,), jnp.int32),
        pltpu.VMEM((STRIPE, PDIM), jnp.float32),
        pltpu.VMEM((STRIPE, PDIM), jnp.float32),
        pltpu.VMEM((8, 8, PDIM), jnp.float32),
        pltpu.VMEM((8, 8, PDIM), jnp.float32),
        pltpu.SemaphoreType.DMA,
        pltpu.SemaphoreType.DMA,
        pltpu.SemaphoreType.DMA,
        pltpu.SemaphoreType.DMA,
    ],
    compiler_params=pltpu.CompilerParams(needs_layout_passes=False),
)
def _gather_kernel(x_hbm, tab_hbm, out_hbm,
                   xblk, kb0, kb1, cr0, cr1, tb0, tb1,
                   gsem0, gsem1, osem0, osem1):
    wid = lax.axis_index("s") * NC + lax.axis_index("c")
    iota = lax.iota(jnp.int32, 16)
    bufs = ((kb0, cr0, tb0, gsem0, osem0), (kb1, cr1, tb1, gsem1, osem1))

    def build_idx_and_fire(kb, colrows, gsem, j):
        jv = jnp.full((16,), 0, jnp.int32) + j
        for g in range(STRIPE // 16):
            kb[pl.ds(g * 16, 16)] = plsc.load_gather(
                xblk, [g * 16 + iota, jv])
        pltpu.async_copy(tab_hbm.at[kb], colrows, gsem)

    rowvecs = [g * 16 + iota for g in range(8)]

    def transpose_and_write(kb, colrows, tbuf, gsem, osem, ib, j):
        pltpu.make_async_copy(tab_hbm.at[kb], colrows, gsem).wait()
        for cb in range(8):
            for c8 in range(8):
                cvec = jnp.full((16,), cb * 8 + c8, jnp.int32)
                for g in range(8):
                    tbuf[cb, c8, pl.ds(g * 16, 16)] = plsc.load_gather(
                        colrows, [rowvecs[g], cvec])
            pltpu.async_copy(tbuf.at[cb], out_hbm.at[j, cb, ib], osem)

    def drain_tiles(tbuf, osem, ib, j):
        for cb in range(8):
            pltpu.make_async_copy(tbuf.at[cb], out_hbm.at[j, cb, ib],
                                  osem).wait()

    def per_stripe(s, carry):
        ib = wid * STRIPES_PER_W + s
        pltpu.sync_copy(x_hbm.at[pl.ds(ib * STRIPE, STRIPE)], xblk)
        for b in range(NBUF):
            kb, colrows, _, gsem, _ = bufs[b]
            build_idx_and_fire(kb, colrows, gsem, b)

        def per_pair(p, c):
            for b in range(NBUF):
                j = p * NBUF + b
                kb, colrows, tbuf, gsem, osem = bufs[b]

                @pl.when(j >= NBUF)
                def _():
                    drain_tiles(tbuf, osem, ib, j - NBUF)

                transpose_and_write(kb, colrows, tbuf, gsem, osem, ib, j)

                @pl.when(j + NBUF < B_COLS)
                def _():
                    build_idx_and_fire(kb, colrows, gsem, j + NBUF)

            return c

        lax.fori_loop(0, B_COLS // NBUF, per_pair, 0)
        for b in range(NBUF):
            _, _, tbuf, _, osem = bufs[b]
            drain_tiles(tbuf, osem, ib, B_COLS - NBUF + b)
        return carry

    lax.fori_loop(0, STRIPES_PER_W, per_stripe, 0)


def kernel(x, emb_weight):
    tab_pad = jnp.pad(emb_weight, ((0, 0), (0, PDIM - DIM)))
    out5d = _gather_kernel(x, tab_pad)
    return jnp.transpose(out5d, (2, 4, 0, 1, 3)).reshape(B_ROWS, B_COLS, DIM)


# speedup vs baseline: 1.4695x; 1.4695x over previous
import functools

import jax
import jax.numpy as jnp
from jax import lax
from jax.experimental import pallas as pl
from jax.experimental.pallas import tpu as pltpu
from jax.experimental.pallas import tpu_sc as plsc

B_ROWS = 16384
B_COLS = 26
DIM = 64
PDIM = 128

_info = plsc.get_sparse_core_info()
NC = _info.num_cores
NS = _info.num_subcores
NW = NC * NS
ROWS_PER_W = B_ROWS // NW
RCHUNK = 8
NIDX = RCHUNK * B_COLS
N_CHUNKS = ROWS_PER_W // RCHUNK
NBUF = 2

_mesh = plsc.VectorSubcoreMesh(core_axis_name="c", subcore_axis_name="s")


@functools.partial(
    pl.kernel,
    out_type=jax.ShapeDtypeStruct((B_ROWS, B_COLS, DIM), jnp.float32),
    mesh=_mesh,
    scratch_types=[
        pltpu.VMEM((RCHUNK, B_COLS), jnp.int32),
        pltpu.VMEM((RCHUNK, B_COLS), jnp.int32),
        pltpu.VMEM((NIDX,), jnp.int32),
        pltpu.VMEM((NIDX,), jnp.int32),
        pltpu.VMEM((NIDX, PDIM), jnp.float32),
        pltpu.VMEM((NIDX, PDIM), jnp.float32),
        pltpu.VMEM((NIDX, DIM), jnp.float32),
        pltpu.VMEM((NIDX, DIM), jnp.float32),
        pltpu.SemaphoreType.DMA,
        pltpu.SemaphoreType.DMA,
        pltpu.SemaphoreType.DMA,
        pltpu.SemaphoreType.DMA,
    ],
)
def _gather_kernel(x_hbm, tab_hbm, out_hbm,
                   xv0, xv1, kb0, kb1, rw0, rw1, pk0, pk1,
                   gsem0, gsem1, osem0, osem1):
    wid = lax.axis_index("s") * NC + lax.axis_index("c")
    base = wid * ROWS_PER_W
    bufs = ((xv0, kb0, rw0, pk0, gsem0, osem0),
            (xv1, kb1, rw1, pk1, gsem1, osem1))

    def stage_and_fire(xv, kb, rows, gsem, row0):
        pltpu.sync_copy(x_hbm.at[pl.ds(row0, RCHUNK)], xv)
        for r in range(RCHUNK):
            kb[pl.ds(r * B_COLS, 16)] = xv[r, pl.ds(0, 16)]
            kb[pl.ds(r * B_COLS + B_COLS - 16, 16)] = \
                xv[r, pl.ds(B_COLS - 16, 16)]
        pltpu.async_copy(tab_hbm.at[kb], rows, gsem)

    def compact(rows, packed, r):
        for j in range(B_COLS):
            n = r * B_COLS + j
            for q in range(DIM // 16):
                packed[n, pl.ds(q * 16, 16)] = rows[n, pl.ds(q * 16, 16)]

    def wait_writeback(packed, osem, row0):
        for r in range(RCHUNK):
            pltpu.make_async_copy(packed.at[pl.ds(r * B_COLS, B_COLS)],
                                  out_hbm.at[row0 + r], osem).wait()

    for b in range(NBUF):
        xv, kb, rows, _, gsem, _ = bufs[b]
        stage_and_fire(xv, kb, rows, gsem, base + b * RCHUNK)

    def outer(j, carry):
        for b in range(NBUF):
            i = j * NBUF + b
            xv, kb, rows, packed, gsem, osem = bufs[b]
            row0 = base + i * RCHUNK
            pltpu.make_async_copy(tab_hbm.at[kb], rows, gsem).wait()

            @pl.when(i >= NBUF)
            def _():
                wait_writeback(packed, osem, row0 - NBUF * RCHUNK)

            def crun(r, c):
                compact(rows, packed, r)
                return c

            lax.fori_loop(0, RCHUNK, crun, 0)
            for r in range(RCHUNK):
                pltpu.async_copy(packed.at[pl.ds(r * B_COLS, B_COLS)],
                                 out_hbm.at[row0 + r], osem)

            @pl.when(i + NBUF < N_CHUNKS)
            def _():
                stage_and_fire(xv, kb, rows, gsem,
                               base + (i + NBUF) * RCHUNK)

        return carry

    lax.fori_loop(0, N_CHUNKS // NBUF, outer, 0)

    for b in range(NBUF):
        _, _, _, packed, _, osem = bufs[b]
        i = N_CHUNKS - NBUF + b
        wait_writeback(packed, osem, base + i * RCHUNK)


def kernel(x, emb_weight):
    tab_pad = jnp.pad(emb_weight.T, ((0, PDIM - DIM), (0, 0))).T
    return _gather_kernel(x, tab_pad)


# speedup vs baseline: 1.6572x; 1.1277x over previous
import functools

import jax
import jax.numpy as jnp
from jax import lax
from jax.experimental import pallas as pl
from jax.experimental.pallas import tpu as pltpu
from jax.experimental.pallas import tpu_sc as plsc

B_ROWS = 16384
B_COLS = 26
DIM = 64
PDIM = 128

_info = plsc.get_sparse_core_info()
NC = _info.num_cores
NS = _info.num_subcores
NW = NC * NS
ROWS_PER_W = B_ROWS // NW
RCHUNK = 8
NIDX = RCHUNK * B_COLS
N_CHUNKS = ROWS_PER_W // RCHUNK
NBUF = 2

_mesh = plsc.VectorSubcoreMesh(core_axis_name="c", subcore_axis_name="s")


@functools.partial(
    pl.kernel,
    out_type=jax.ShapeDtypeStruct((B_ROWS, B_COLS, DIM), jnp.float32),
    mesh=_mesh,
    scratch_types=[
        pltpu.VMEM((RCHUNK, B_COLS), jnp.int32),
        pltpu.VMEM((RCHUNK, B_COLS), jnp.int32),
        pltpu.VMEM((NIDX,), jnp.int32),
        pltpu.VMEM((NIDX,), jnp.int32),
        pltpu.VMEM((NIDX, PDIM), jnp.float32),
        pltpu.VMEM((NIDX, PDIM), jnp.float32),
        pltpu.VMEM((NIDX, DIM), jnp.float32),
        pltpu.VMEM((NIDX, DIM), jnp.float32),
        pltpu.SemaphoreType.DMA,
        pltpu.SemaphoreType.DMA,
        pltpu.SemaphoreType.DMA,
        pltpu.SemaphoreType.DMA,
    ],
)
def _gather_kernel(x_hbm, tab_hbm, out_hbm,
                   xv0, xv1, kb0, kb1, rw0, rw1, pk0, pk1,
                   gsem0, gsem1, osem0, osem1):
    wid = lax.axis_index("s") * NC + lax.axis_index("c")
    base = wid * ROWS_PER_W
    bufs = ((xv0, kb0, rw0, pk0, gsem0, osem0),
            (xv1, kb1, rw1, pk1, gsem1, osem1))

    def stage_and_fire(xv, kb, rows, gsem, row0):
        pltpu.sync_copy(x_hbm.at[pl.ds(row0, RCHUNK)], xv)
        for r in range(RCHUNK):
            kb[pl.ds(r * B_COLS, 16)] = xv[r, pl.ds(0, 16)]
            kb[pl.ds(r * B_COLS + B_COLS - 16, 16)] = \
                xv[r, pl.ds(B_COLS - 16, 16)]
        pltpu.async_copy(tab_hbm.at[kb], rows, gsem)

    def compact(rows, packed, r):
        for j in range(B_COLS):
            n = r * B_COLS + j
            for q in range(DIM // 16):
                packed[n, pl.ds(q * 16, 16)] = rows[n, pl.ds(q * 16, 16)]

    def wait_writeback(packed, osem, row0):
        for r in range(RCHUNK):
            pltpu.make_async_copy(packed.at[pl.ds(r * B_COLS, B_COLS)],
                                  out_hbm.at[row0 + r], osem).wait()

    for b in range(NBUF):
        xv, kb, rows, _, gsem, _ = bufs[b]
        stage_and_fire(xv, kb, rows, gsem, base + b * RCHUNK)

    def outer(j, carry):
        for b in range(NBUF):
            i = j * NBUF + b
            xv, kb, rows, packed, gsem, osem = bufs[b]
            row0 = base + i * RCHUNK
            pltpu.make_async_copy(tab_hbm.at[kb], rows, gsem).wait()

            @pl.when(i >= NBUF)
            def _():
                wait_writeback(packed, osem, row0 - NBUF * RCHUNK)

            def crun(r, c):
                compact(rows, packed, r)
                return c

            lax.fori_loop(0, RCHUNK, crun, 0)
            for r in range(RCHUNK):
                pltpu.async_copy(packed.at[pl.ds(r * B_COLS, B_COLS)],
                                 out_hbm.at[row0 + r], osem)

            @pl.when(i + NBUF < N_CHUNKS)
            def _():
                stage_and_fire(xv, kb, rows, gsem,
                               base + (i + NBUF) * RCHUNK)

        return carry

    lax.fori_loop(0, N_CHUNKS // NBUF, outer, 0)

    for b in range(NBUF):
        _, _, _, packed, _, osem = bufs[b]
        i = N_CHUNKS - NBUF + b
        wait_writeback(packed, osem, base + i * RCHUNK)


def kernel(x, emb_weight):
    tab_pad = jnp.pad(emb_weight, ((0, 0), (0, PDIM - DIM)))
    out = _gather_kernel(x, tab_pad)
    return lax.optimization_barrier(out)
